# baseline (speedup 1.0000x reference)
import functools

import numpy as np
import jax
import jax.numpy as jnp
from jax.experimental import pallas as pl

_B = 256
_N = 1024
_NUM_MASK = 768
_TAIL = _N - _NUM_MASK


def _threefry_bits_u32(n):
    k0 = jnp.uint32(0)
    k1 = jnp.uint32(1)
    k2 = jnp.uint32(0x1BD11BDA) ^ k0 ^ k1
    ks = (k0, k1, k2)
    rot = ((13, 15, 26, 6), (17, 29, 16, 24))
    x0 = jnp.zeros_like(n) + k0
    x1 = n + k1
    for i in range(5):
        for r in rot[i % 2]:
            x0 = x0 + x1
            x1 = (x1 << jnp.uint32(r)) | (x1 >> jnp.uint32(32 - r))
            x1 = x0 ^ x1
        x0 = x0 + ks[(i + 1) % 3]
        x1 = x1 + ks[(i + 2) % 3] + jnp.uint32(i + 1)
    return x0 ^ x1


_ROWS_PER_STEP = 8


def _row_body(eq768_ref, lt768_ref, out_ref):
    step = pl.program_id(0)
    eq768 = eq768_ref[...]
    lt768 = lt768_ref[...]
    ones_row = jnp.ones((1, _N), jnp.float32)
    ones_col = jnp.ones((1, _TAIL), jnp.float32)
    j = jax.lax.broadcasted_iota(jnp.uint32, (1, _N), 1)
    jf = jax.lax.broadcasted_iota(jnp.int32, (1, _N), 1).astype(jnp.float32)

    for rr in range(_ROWS_PER_STEP):
        r = step * _ROWS_PER_STEP + rr
        n = (r * _N).astype(jnp.uint32) + j
        bits = _threefry_bits_u32(n)
        hif = (bits >> jnp.uint32(9)).astype(jnp.float32)

        t_col = jax.lax.dot_general(
            eq768, hif, (((1,), (1,)), ((), ())),
            preferred_element_type=jnp.float32)

        ltm = (hif < t_col).astype(jnp.float32)
        eqm = (hif == t_col).astype(jnp.float32)
        contrib = ltm + eqm * lt768

        rank_col = jax.lax.dot_general(
            contrib, ones_row, (((1,), (1,)), ((), ())),
            preferred_element_type=jnp.float32)

        eq2 = (rank_col == jf).astype(jnp.float32)
        notmask = jax.lax.dot_general(
            ones_col, eq2, (((1,), (0,)), ((), ())),
            preferred_element_type=jnp.float32)

        out_ref[pl.ds(rr, 1), :] = notmask == 0.0


@functools.partial(jax.jit, static_argnames=("interpret",))
def _random_mask(interpret=False):
    a = np.arange(_TAIL)[:, None] + _NUM_MASK
    jj = np.arange(_N)[None, :]
    eq768 = (jj == a).astype(np.float32)
    lt768 = (jj < a).astype(np.float32)
    return pl.pallas_call(
        _row_body,
        grid=(_B // _ROWS_PER_STEP,),
        in_specs=[
            pl.BlockSpec((_TAIL, _N), lambda r: (0, 0)),
            pl.BlockSpec((_TAIL, _N), lambda r: (0, 0)),
        ],
        out_specs=pl.BlockSpec((_ROWS_PER_STEP, _N), lambda r: (r, 0)),
        out_shape=jax.ShapeDtypeStruct((_B, _N), jnp.bool_),
        interpret=interpret,
    )(jnp.asarray(eq768), jnp.asarray(lt768))


def kernel(x):
    assert x.shape[0] == _B
    return _random_mask()


# speedup vs baseline: 1.5833x; 1.5833x over previous
import functools

import numpy as np
import jax
import jax.numpy as jnp
from jax.experimental import pallas as pl

_B = 256
_N = 1024
_NUM_MASK = 768
_TAIL = _N - _NUM_MASK
_R = 8


def _threefry_bits_u32(n):
    k0 = jnp.uint32(0)
    k1 = jnp.uint32(1)
    k2 = jnp.uint32(0x1BD11BDA) ^ k0 ^ k1
    ks = (k0, k1, k2)
    rot = ((13, 15, 26, 6), (17, 29, 16, 24))
    x0 = jnp.zeros_like(n) + k0
    x1 = n + k1
    for i in range(5):
        for r in rot[i % 2]:
            x0 = x0 + x1
            x1 = (x1 << jnp.uint32(r)) | (x1 >> jnp.uint32(32 - r))
            x1 = x0 ^ x1
        x0 = x0 + ks[(i + 1) % 3]
        x1 = x1 + ks[(i + 2) % 3] + jnp.uint32(i + 1)
    return x0 ^ x1


def _body(ones_ref, out_ref):
    step = pl.program_id(0)

    j = jax.lax.broadcasted_iota(jnp.uint32, (_R, _N), 1)
    si = jax.lax.broadcasted_iota(jnp.uint32, (_R, _N), 0)
    n = (step * (_R * _N)).astype(jnp.uint32) + si * jnp.uint32(_N) + j
    bits = _threefry_bits_u32(n)

    k = (bits & jnp.uint32(0xFFFFFE00)) | (j >> jnp.uint32(1))
    k = k ^ jnp.uint32(0x80000000)
    kxi = jax.lax.bitcast_convert_type(k, jnp.int32)

    tail = kxi[:, _NUM_MASK:]
    tt_i = jnp.transpose(tail, (1, 0))

    ones_row = ones_ref[...]
    ch_iota = jax.lax.broadcasted_iota(jnp.int32, (_TAIL, 32), 1)

    for rr in range(_R):
        t_col = jax.lax.slice(tt_i, (0, rr), (_TAIL, rr + 1))
        krow = kxi[rr:rr + 1, :]
        cmpf = (krow < t_col).astype(jnp.float32)
        rank_col = jax.lax.dot_general(
            cmpf, ones_row, (((1,), (1,)), ((), ())),
            preferred_element_type=jnp.float32)

        r_i = rank_col.astype(jnp.int32)
        rhi = r_i >> 5
        rlo = r_i & 31
        u = (rhi == ch_iota).astype(jnp.float32)
        v = (rlo == ch_iota).astype(jnp.float32)
        s = jax.lax.dot_general(
            u, v, (((0,), (0,)), ((), ())),
            preferred_element_type=jnp.float32)
        out_ref[rr] = s == 0.0


@functools.partial(jax.jit, static_argnames=("interpret",))
def _random_mask(interpret=False):
    ones = np.ones((1, _N), dtype=np.float32)
    blocks = pl.pallas_call(
        _body,
        grid=(_B // _R,),
        in_specs=[
            pl.BlockSpec((1, _N), lambda r: (0, 0)),
        ],
        out_specs=pl.BlockSpec((_R, 32, 32), lambda r: (r, 0, 0)),
        out_shape=jax.ShapeDtypeStruct((_B, 32, 32), jnp.bool_),
        interpret=interpret,
    )(jnp.asarray(ones))
    return blocks.reshape(_B, _N)


def kernel(x):
    assert x.shape[0] == _B
    return _random_mask()
